# baseline (speedup 1.0000x reference)
import functools

import jax
import jax.numpy as jnp
from jax import lax
from jax.experimental import pallas as pl
from jax.experimental.pallas import tpu as pltpu
from jax.experimental.pallas import tpu_sc as plsc

N = 10000
D = 128
CD = 64
E = 320000
NMOTIF = 13
NG = NMOTIF + 1

NCORES = 2
NSUB = 16
NTILES = NCORES * NSUB
CHUNK = 128
RUN_CHUNKS = 16
PAIRS = RUN_CHUNKS // 2
RUNS = 5
NCHUNKS = RUNS * RUN_CHUNKS
EPT = NCHUNKS * CHUNK
EPAD = EPT * NTILES
NP = 10240
RPT = NP // NSUB


def _conv_body(ng, xx, src, dst, ew, out, src_b, dst_b, ew_b, rows0, rows1,
               zero_v, acc, sg0, sg1, ss0, ss1):
    cid = lax.axis_index("c")
    sid = lax.axis_index("s")
    cbase = (cid * NSUB + sid) * NCHUNKS

    z16 = jnp.zeros((16,), jnp.float32)
    for r in range(16):
        for c in range(D // 16):
            zero_v[r, pl.ds(c * 16, 16)] = z16

    def scale(rv, wrow):
        def group_body(gi, carry3):
            wg = ew_b[wrow, pl.ds(gi * 16, 16)]
            for t in range(16):
                e = gi * 16 + t
                w = lax.gather(
                    wg, jnp.full((16, 1), t, jnp.int32),
                    lax.GatherDimensionNumbers(
                        offset_dims=(), collapsed_slice_dims=(0,),
                        start_index_map=(0,)),
                    slice_sizes=(1,),
                    mode=lax.GatherScatterMode.PROMISE_IN_BOUNDS)
                for c in range(D // 16):
                    rv[e, pl.ds(c * 16, 16)] = rv[e, pl.ds(c * 16, 16)] * w
            return carry3

        lax.fori_loop(0, CHUNK // 16, group_body, 0)

    def graph_body(g, carry):
        def zero_body(z, carry0):
            pltpu.sync_copy(zero_v, acc.at[pl.ds(sid * RPT + z * 16, 16)])
            return carry0

        lax.fori_loop(0, RPT // 16, zero_body, 0)
        plsc.subcore_barrier()

        def run_body(r, carry1):
            rb = cbase + r * RUN_CHUNKS
            pltpu.sync_copy(src.at[g, pl.ds(rb, RUN_CHUNKS)], src_b)
            pltpu.sync_copy(dst.at[g, pl.ds(rb, RUN_CHUNKS)], dst_b)
            pltpu.sync_copy(ew.at[g, pl.ds(rb, RUN_CHUNKS)], ew_b)
            pltpu.async_copy(xx.at[src_b.at[0]], rows0, sg0)

            def pair_body(p, carry2):
                j0 = 2 * p
                j1 = 2 * p + 1

                @pl.when(p > 0)
                def _wait_s1():
                    pltpu.make_async_copy(rows1, acc.at[dst_b.at[j1]],
                                          ss1).wait()

                pltpu.async_copy(xx.at[src_b.at[j1]], rows1, sg1)
                pltpu.make_async_copy(xx.at[src_b.at[j0]], rows0, sg0).wait()
                scale(rows0, j0)
                pltpu.async_copy(rows0, acc.at[dst_b.at[j0]], ss0, add=True)
                pltpu.make_async_copy(xx.at[src_b.at[j1]], rows1, sg1).wait()
                scale(rows1, j1)
                pltpu.make_async_copy(rows0, acc.at[dst_b.at[j0]], ss0).wait()

                @pl.when(p < PAIRS - 1)
                def _next_g0():
                    pltpu.async_copy(xx.at[src_b.at[j0 + 2]], rows0, sg0)

                pltpu.async_copy(rows1, acc.at[dst_b.at[j1]], ss1, add=True)
                return carry2

            lax.fori_loop(0, PAIRS, pair_body, 0)
            pltpu.make_async_copy(rows1, acc.at[dst_b.at[RUN_CHUNKS - 1]],
                                  ss1).wait()
            return carry1

        lax.fori_loop(0, RUNS, run_body, 0)
        plsc.subcore_barrier()
        pltpu.sync_copy(acc.at[pl.ds(sid * RPT, RPT)],
                        out.at[cid, g, pl.ds(sid * RPT, RPT)])
        return carry

    lax.fori_loop(0, ng, graph_body, 0)


@functools.lru_cache(maxsize=None)
def _make_conv(ng):
    mesh = plsc.VectorSubcoreMesh(core_axis_name="c", subcore_axis_name="s")
    return pl.kernel(
        functools.partial(_conv_body, ng),
        out_type=jax.ShapeDtypeStruct((NCORES, ng, NP, D), jnp.float32),
        mesh=mesh,
        scratch_types=[
            pltpu.VMEM((RUN_CHUNKS, CHUNK), jnp.int32),
            pltpu.VMEM((RUN_CHUNKS, CHUNK), jnp.int32),
            pltpu.VMEM((RUN_CHUNKS, CHUNK), jnp.float32),
            pltpu.VMEM((CHUNK, D), jnp.float32),
            pltpu.VMEM((CHUNK, D), jnp.float32),
            pltpu.VMEM((16, D), jnp.float32),
            pltpu.VMEM_SHARED((NP, D), jnp.float32),
            pltpu.SemaphoreType.DMA,
            pltpu.SemaphoreType.DMA,
            pltpu.SemaphoreType.DMA,
            pltpu.SemaphoreType.DMA,
        ],
    )


R_H = 1000


def _h_body(p_ref, x_ref, w_ref, r_ref, b_ref, o_ref):
    agg = p_ref[0] + p_ref[1]
    o_ref[...] = (
        jnp.dot(agg, w_ref[...], preferred_element_type=jnp.float32)
        + jnp.dot(x_ref[...], r_ref[...], preferred_element_type=jnp.float32)
        + b_ref[...]
    )


R_A = 200


def _att_body(h_ref, p_ref, wb_ref, bc_ref, wa_ref, ba_ref, o_ref):
    f32 = jnp.float32
    wa = wa_ref[...]
    c = jnp.dot(h_ref[...], wb_ref[0:D], preferred_element_type=f32)
    mws = []
    for j in range(NMOTIF):
        mj = p_ref[0, j] + p_ref[1, j]
        c = c + jnp.dot(mj, wb_ref[D * (j + 1):D * (j + 2)],
                        preferred_element_type=f32)
        mws.append(jnp.dot(mj, wa, preferred_element_type=f32))
    c = c + bc_ref[...]
    mw = jnp.concatenate(mws, axis=1) + ba_ref[...]
    att = jnp.tanh(jnp.sum((mw * c).reshape(R_A, NMOTIF, CD), axis=2))
    diff = (mw - c).reshape(R_A, NMOTIF, CD)
    o_ref[...] = (att[:, :, None] * diff).reshape(R_A, NMOTIF * CD)


def _build_wbig(motif_w):
    cols = []
    zblk = jnp.zeros((D, CD), jnp.float32)
    for i in range(1, NMOTIF + 1):
        wm = motif_w[i - 1]
        cols.append(jnp.concatenate([wm[: i * D], zblk, wm[i * D:]], axis=0))
    return jnp.concatenate(cols, axis=1)


def kernel(x, edge_index, edge_weight, weight, root, bias, wa, ba, motif_w,
           motif_b):
    nchunk_tot = EPAD // CHUNK
    src = jnp.pad(edge_index[:, 0, :],
                  ((0, 0), (0, EPAD - E))).reshape(NG, nchunk_tot, CHUNK)
    dst = jnp.pad(edge_index[:, 1, :],
                  ((0, 0), (0, EPAD - E))).reshape(NG, nchunk_tot, CHUNK)
    ew = jnp.pad(edge_weight,
                 ((0, 0), (0, EPAD - E))).reshape(NG, nchunk_tot, CHUNK)

    p0 = _make_conv(1)(x, src[:1], dst[:1], ew[:1])[:, 0, :N, :]

    h = pl.pallas_call(
        _h_body,
        grid=(N // R_H,),
        in_specs=[
            pl.BlockSpec((NCORES, R_H, D), lambda i: (0, i, 0)),
            pl.BlockSpec((R_H, D), lambda i: (i, 0)),
            pl.BlockSpec((D, D), lambda i: (0, 0)),
            pl.BlockSpec((D, D), lambda i: (0, 0)),
            pl.BlockSpec((1, D), lambda i: (0, 0)),
        ],
        out_specs=pl.BlockSpec((R_H, D), lambda i: (i, 0)),
        out_shape=jax.ShapeDtypeStruct((N, D), jnp.float32),
    )(p0, x, weight, root, bias[None, :])

    p = _make_conv(NMOTIF)(h, src[1:], dst[1:], ew[1:])[:, :, :N, :]

    wbig = _build_wbig(motif_w)
    bcat = motif_b.reshape(1, NMOTIF * CD)
    batile = jnp.tile(ba, NMOTIF)[None, :]

    out = pl.pallas_call(
        _att_body,
        grid=(N // R_A,),
        in_specs=[
            pl.BlockSpec((R_A, D), lambda i: (i, 0)),
            pl.BlockSpec((NCORES, NMOTIF, R_A, D), lambda i: (0, 0, i, 0)),
            pl.BlockSpec(((NMOTIF + 1) * D, NMOTIF * CD), lambda i: (0, 0)),
            pl.BlockSpec((1, NMOTIF * CD), lambda i: (0, 0)),
            pl.BlockSpec((D, CD), lambda i: (0, 0)),
            pl.BlockSpec((1, NMOTIF * CD), lambda i: (0, 0)),
        ],
        out_specs=pl.BlockSpec((R_A, NMOTIF * CD), lambda i: (i, 0)),
        out_shape=jax.ShapeDtypeStruct((N, NMOTIF * CD), jnp.float32),
    )(h, p, wbig, bcat, wa, batile)
    return out


# speedup vs baseline: 1.1282x; 1.1282x over previous
import functools

import jax
import jax.numpy as jnp
from jax import lax
from jax.experimental import pallas as pl
from jax.experimental.pallas import tpu as pltpu
from jax.experimental.pallas import tpu_sc as plsc

N = 10000
D = 128
CD = 64
E = 320000
NMOTIF = 13
NG = NMOTIF + 1

NCORES = 2
NSUB = 16
NTILES = NCORES * NSUB
CHUNK = 128
RUN_CHUNKS = 8
PAIRS = RUN_CHUNKS // 2
CPC = (120, 40)
NCHUNKS = CPC[0] + CPC[1]
EPAD = NCHUNKS * NSUB * CHUNK
NP = 10240
RPT = NP // NSUB


def _conv_body(ng, xx, src, dst, ew, out, src_b, dst_b, ew_b, rows0, rows1,
               zero_v, acc, sg0, sg1, ss0, ss1):
    cid = lax.axis_index("c")
    sid = lax.axis_index("s")
    cbase = jnp.where(cid == 0, sid * CPC[0], NSUB * CPC[0] + sid * CPC[1])
    nruns = jnp.where(cid == 0, CPC[0] // RUN_CHUNKS, CPC[1] // RUN_CHUNKS)

    z16 = jnp.zeros((16,), jnp.float32)
    for r in range(16):
        for c in range(D // 16):
            zero_v[r, pl.ds(c * 16, 16)] = z16

    def scale(rv, wrow):
        def group_body(gi, carry3):
            wg = ew_b[wrow, pl.ds(gi * 16, 16)]
            for t in range(16):
                e = gi * 16 + t
                w = lax.gather(
                    wg, jnp.full((16, 1), t, jnp.int32),
                    lax.GatherDimensionNumbers(
                        offset_dims=(), collapsed_slice_dims=(0,),
                        start_index_map=(0,)),
                    slice_sizes=(1,),
                    mode=lax.GatherScatterMode.PROMISE_IN_BOUNDS)
                for c in range(D // 16):
                    rv[e, pl.ds(c * 16, 16)] = rv[e, pl.ds(c * 16, 16)] * w
            return carry3

        lax.fori_loop(0, CHUNK // 16, group_body, 0)

    def graph_body(g, carry):
        def zero_body(z, carry0):
            pltpu.sync_copy(zero_v, acc.at[pl.ds(sid * RPT + z * 16, 16)])
            return carry0

        lax.fori_loop(0, RPT // 16, zero_body, 0)
        plsc.subcore_barrier()

        def run_body(r, carry1):
            rb = cbase + r * RUN_CHUNKS
            pltpu.sync_copy(src.at[g, pl.ds(rb, RUN_CHUNKS)], src_b)
            pltpu.sync_copy(dst.at[g, pl.ds(rb, RUN_CHUNKS)], dst_b)
            pltpu.sync_copy(ew.at[g, pl.ds(rb, RUN_CHUNKS)], ew_b)
            pltpu.async_copy(xx.at[src_b.at[0]], rows0, sg0)

            def pair_body(p, carry2):
                j0 = 2 * p
                j1 = 2 * p + 1

                @pl.when(p > 0)
                def _wait_s1():
                    pltpu.make_async_copy(rows1, acc.at[dst_b.at[j1]],
                                          ss1).wait()

                pltpu.async_copy(xx.at[src_b.at[j1]], rows1, sg1)
                pltpu.make_async_copy(xx.at[src_b.at[j0]], rows0, sg0).wait()
                scale(rows0, j0)
                pltpu.async_copy(rows0, acc.at[dst_b.at[j0]], ss0, add=True)
                pltpu.make_async_copy(xx.at[src_b.at[j1]], rows1, sg1).wait()
                scale(rows1, j1)
                pltpu.make_async_copy(rows0, acc.at[dst_b.at[j0]], ss0).wait()

                @pl.when(p < PAIRS - 1)
                def _next_g0():
                    pltpu.async_copy(xx.at[src_b.at[j0 + 2]], rows0, sg0)

                pltpu.async_copy(rows1, acc.at[dst_b.at[j1]], ss1, add=True)
                return carry2

            lax.fori_loop(0, PAIRS, pair_body, 0)
            pltpu.make_async_copy(rows1, acc.at[dst_b.at[RUN_CHUNKS - 1]],
                                  ss1).wait()
            return carry1

        lax.fori_loop(0, nruns, run_body, 0)
        plsc.subcore_barrier()
        pltpu.sync_copy(acc.at[pl.ds(sid * RPT, RPT)],
                        out.at[cid, g, pl.ds(sid * RPT, RPT)])
        return carry

    lax.fori_loop(0, ng, graph_body, 0)


@functools.lru_cache(maxsize=None)
def _make_conv(ng):
    mesh = plsc.VectorSubcoreMesh(core_axis_name="c", subcore_axis_name="s")
    return pl.kernel(
        functools.partial(_conv_body, ng),
        out_type=jax.ShapeDtypeStruct((NCORES, ng, NP, D), jnp.float32),
        mesh=mesh,
        scratch_types=[
            pltpu.VMEM((RUN_CHUNKS, CHUNK), jnp.int32),
            pltpu.VMEM((RUN_CHUNKS, CHUNK), jnp.int32),
            pltpu.VMEM((RUN_CHUNKS, CHUNK), jnp.float32),
            pltpu.VMEM((CHUNK, D), jnp.float32),
            pltpu.VMEM((CHUNK, D), jnp.float32),
            pltpu.VMEM((16, D), jnp.float32),
            pltpu.VMEM_SHARED((NP, D), jnp.float32),
            pltpu.SemaphoreType.DMA,
            pltpu.SemaphoreType.DMA,
            pltpu.SemaphoreType.DMA,
            pltpu.SemaphoreType.DMA,
        ],
    )


R_H = 1000


def _h_body(p_ref, x_ref, w_ref, r_ref, b_ref, o_ref):
    agg = p_ref[0] + p_ref[1]
    o_ref[...] = (
        jnp.dot(agg, w_ref[...], preferred_element_type=jnp.float32)
        + jnp.dot(x_ref[...], r_ref[...], preferred_element_type=jnp.float32)
        + b_ref[...]
    )


R_A = 200


def _att_body(h_ref, p_ref, wb_ref, bc_ref, wa_ref, ba_ref, o_ref):
    f32 = jnp.float32
    wa = wa_ref[...]
    c = jnp.dot(h_ref[...], wb_ref[0:D], preferred_element_type=f32)
    mws = []
    for j in range(NMOTIF):
        mj = p_ref[0, j] + p_ref[1, j]
        c = c + jnp.dot(mj, wb_ref[D * (j + 1):D * (j + 2)],
                        preferred_element_type=f32)
        mws.append(jnp.dot(mj, wa, preferred_element_type=f32))
    c = c + bc_ref[...]
    mw = jnp.concatenate(mws, axis=1) + ba_ref[...]
    att = jnp.tanh(jnp.sum((mw * c).reshape(R_A, NMOTIF, CD), axis=2))
    diff = (mw - c).reshape(R_A, NMOTIF, CD)
    o_ref[...] = (att[:, :, None] * diff).reshape(R_A, NMOTIF * CD)


def _build_wbig(motif_w):
    cols = []
    zblk = jnp.zeros((D, CD), jnp.float32)
    for i in range(1, NMOTIF + 1):
        wm = motif_w[i - 1]
        cols.append(jnp.concatenate([wm[: i * D], zblk, wm[i * D:]], axis=0))
    return jnp.concatenate(cols, axis=1)


def kernel(x, edge_index, edge_weight, weight, root, bias, wa, ba, motif_w,
           motif_b):
    nchunk_tot = EPAD // CHUNK
    src = jnp.pad(edge_index[:, 0, :],
                  ((0, 0), (0, EPAD - E))).reshape(NG, nchunk_tot, CHUNK)
    dst = jnp.pad(edge_index[:, 1, :],
                  ((0, 0), (0, EPAD - E))).reshape(NG, nchunk_tot, CHUNK)
    ew = jnp.pad(edge_weight,
                 ((0, 0), (0, EPAD - E))).reshape(NG, nchunk_tot, CHUNK)

    p0 = _make_conv(1)(x, src[:1], dst[:1], ew[:1])[:, 0, :N, :]

    h = pl.pallas_call(
        _h_body,
        grid=(N // R_H,),
        in_specs=[
            pl.BlockSpec((NCORES, R_H, D), lambda i: (0, i, 0)),
            pl.BlockSpec((R_H, D), lambda i: (i, 0)),
            pl.BlockSpec((D, D), lambda i: (0, 0)),
            pl.BlockSpec((D, D), lambda i: (0, 0)),
            pl.BlockSpec((1, D), lambda i: (0, 0)),
        ],
        out_specs=pl.BlockSpec((R_H, D), lambda i: (i, 0)),
        out_shape=jax.ShapeDtypeStruct((N, D), jnp.float32),
    )(p0, x, weight, root, bias[None, :])

    p = _make_conv(NMOTIF)(h, src[1:], dst[1:], ew[1:])[:, :, :N, :]

    wbig = _build_wbig(motif_w)
    bcat = motif_b.reshape(1, NMOTIF * CD)
    batile = jnp.tile(ba, NMOTIF)[None, :]

    out = pl.pallas_call(
        _att_body,
        grid=(N // R_A,),
        in_specs=[
            pl.BlockSpec((R_A, D), lambda i: (i, 0)),
            pl.BlockSpec((NCORES, NMOTIF, R_A, D), lambda i: (0, 0, i, 0)),
            pl.BlockSpec(((NMOTIF + 1) * D, NMOTIF * CD), lambda i: (0, 0)),
            pl.BlockSpec((1, NMOTIF * CD), lambda i: (0, 0)),
            pl.BlockSpec((D, CD), lambda i: (0, 0)),
            pl.BlockSpec((1, NMOTIF * CD), lambda i: (0, 0)),
        ],
        out_specs=pl.BlockSpec((R_A, NMOTIF * CD), lambda i: (i, 0)),
        out_shape=jax.ShapeDtypeStruct((N, NMOTIF * CD), jnp.float32),
    )(h, p, wbig, bcat, wa, batile)
    return out


# speedup vs baseline: 1.1283x; 1.0001x over previous
import functools

import jax
import jax.numpy as jnp
from jax import lax
from jax.experimental import pallas as pl
from jax.experimental.pallas import tpu as pltpu
from jax.experimental.pallas import tpu_sc as plsc

N = 10000
D = 128
CD = 64
E = 320000
NMOTIF = 13
NG = NMOTIF + 1

NCORES = 2
NSUB = 16
NTILES = NCORES * NSUB
CHUNK = 128
RUN_CHUNKS = 8
PAIRS = RUN_CHUNKS // 2
CPC = (120, 40)
NCHUNKS = CPC[0] + CPC[1]
EPAD = NCHUNKS * NSUB * CHUNK
NP = 10240
RPT = NP // NSUB


def _conv_body(ng, xx, src, dst, ew, out, src_b, dst_b, ew_b, rows0, rows1,
               zero_v, acc, sg0, sg1, ss0, ss1):
    cid = lax.axis_index("c")
    sid = lax.axis_index("s")
    cbase = jnp.where(cid == 0, sid * CPC[0], NSUB * CPC[0] + sid * CPC[1])
    nruns = jnp.where(cid == 0, CPC[0] // RUN_CHUNKS, CPC[1] // RUN_CHUNKS)

    z16 = jnp.zeros((16,), jnp.float32)
    for r in range(16):
        for c in range(D // 16):
            zero_v[r, pl.ds(c * 16, 16)] = z16

    def scale(rv, wrow):
        def group_body(gi, carry3):
            wg = ew_b[wrow, pl.ds(gi * 16, 16)]
            for t in range(16):
                e = gi * 16 + t
                w = lax.gather(
                    wg, jnp.full((16, 1), t, jnp.int32),
                    lax.GatherDimensionNumbers(
                        offset_dims=(), collapsed_slice_dims=(0,),
                        start_index_map=(0,)),
                    slice_sizes=(1,),
                    mode=lax.GatherScatterMode.PROMISE_IN_BOUNDS)
                for c in range(D // 16):
                    rv[e, pl.ds(c * 16, 16)] = rv[e, pl.ds(c * 16, 16)] * w
            return carry3

        lax.fori_loop(0, CHUNK // 16, group_body, 0)

    def graph_body(g, carry):
        def zero_body(z, carry0):
            pltpu.sync_copy(zero_v, acc.at[pl.ds(sid * RPT + z * 16, 16)])
            return carry0

        with jax.named_scope("acc_zero"):
            lax.fori_loop(0, RPT // 16, zero_body, 0)
            plsc.subcore_barrier()

        def run_body(r, carry1):
            rb = cbase + r * RUN_CHUNKS
            pltpu.sync_copy(src.at[g, pl.ds(rb, RUN_CHUNKS)], src_b)
            pltpu.sync_copy(dst.at[g, pl.ds(rb, RUN_CHUNKS)], dst_b)
            pltpu.sync_copy(ew.at[g, pl.ds(rb, RUN_CHUNKS)], ew_b)
            pltpu.async_copy(xx.at[src_b.at[0]], rows0, sg0)

            def pair_body(p, carry2):
                j0 = 2 * p
                j1 = 2 * p + 1

                @pl.when(p > 0)
                def _wait_s1():
                    pltpu.make_async_copy(rows1, acc.at[dst_b.at[j1]],
                                          ss1).wait()

                pltpu.async_copy(xx.at[src_b.at[j1]], rows1, sg1)
                pltpu.make_async_copy(xx.at[src_b.at[j0]], rows0, sg0).wait()
                scale(rows0, j0)
                pltpu.async_copy(rows0, acc.at[dst_b.at[j0]], ss0, add=True)
                pltpu.make_async_copy(xx.at[src_b.at[j1]], rows1, sg1).wait()
                scale(rows1, j1)
                pltpu.make_async_copy(rows0, acc.at[dst_b.at[j0]], ss0).wait()

                @pl.when(p < PAIRS - 1)
                def _next_g0():
                    pltpu.async_copy(xx.at[src_b.at[j0 + 2]], rows0, sg0)

                pltpu.async_copy(rows1, acc.at[dst_b.at[j1]], ss1, add=True)
                return carry2

            lax.fori_loop(0, PAIRS, pair_body, 0)
            pltpu.make_async_copy(rows1, acc.at[dst_b.at[RUN_CHUNKS - 1]],
                                  ss1).wait()
            return carry1

        with jax.named_scope("edges"):
            lax.fori_loop(0, nruns, run_body, 0)
            plsc.subcore_barrier()
        with jax.named_scope("copyout"):
            pltpu.sync_copy(acc.at[pl.ds(sid * RPT, RPT)],
                            out.at[cid, g, pl.ds(sid * RPT, RPT)])
        return carry

    lax.fori_loop(0, ng, graph_body, 0)


@functools.lru_cache(maxsize=None)
def _make_conv(ng):
    mesh = plsc.VectorSubcoreMesh(core_axis_name="c", subcore_axis_name="s")
    return pl.kernel(
        functools.partial(_conv_body, ng),
        out_type=jax.ShapeDtypeStruct((NCORES, ng, NP, D), jnp.float32),
        mesh=mesh,
        scratch_types=[
            pltpu.VMEM((RUN_CHUNKS, CHUNK), jnp.int32),
            pltpu.VMEM((RUN_CHUNKS, CHUNK), jnp.int32),
            pltpu.VMEM((RUN_CHUNKS, CHUNK), jnp.float32),
            pltpu.VMEM((CHUNK, D), jnp.float32),
            pltpu.VMEM((CHUNK, D), jnp.float32),
            pltpu.VMEM((16, D), jnp.float32),
            pltpu.VMEM_SHARED((NP, D), jnp.float32),
            pltpu.SemaphoreType.DMA,
            pltpu.SemaphoreType.DMA,
            pltpu.SemaphoreType.DMA,
            pltpu.SemaphoreType.DMA,
        ],
    )


R_H = 1000


def _h_body(p_ref, x_ref, w_ref, r_ref, b_ref, o_ref):
    agg = p_ref[0] + p_ref[1]
    o_ref[...] = (
        jnp.dot(agg, w_ref[...], preferred_element_type=jnp.float32)
        + jnp.dot(x_ref[...], r_ref[...], preferred_element_type=jnp.float32)
        + b_ref[...]
    )


R_A = 200


def _att_body(h_ref, p_ref, wb_ref, bc_ref, wa_ref, ba_ref, o_ref):
    f32 = jnp.float32
    wa = wa_ref[...]
    c = jnp.dot(h_ref[...], wb_ref[0:D], preferred_element_type=f32)
    mws = []
    for j in range(NMOTIF):
        mj = p_ref[0, j] + p_ref[1, j]
        c = c + jnp.dot(mj, wb_ref[D * (j + 1):D * (j + 2)],
                        preferred_element_type=f32)
        mws.append(jnp.dot(mj, wa, preferred_element_type=f32))
    c = c + bc_ref[...]
    mw = jnp.concatenate(mws, axis=1) + ba_ref[...]
    att = jnp.tanh(jnp.sum((mw * c).reshape(R_A, NMOTIF, CD), axis=2))
    diff = (mw - c).reshape(R_A, NMOTIF, CD)
    o_ref[...] = (att[:, :, None] * diff).reshape(R_A, NMOTIF * CD)


def _build_wbig(motif_w):
    cols = []
    zblk = jnp.zeros((D, CD), jnp.float32)
    for i in range(1, NMOTIF + 1):
        wm = motif_w[i - 1]
        cols.append(jnp.concatenate([wm[: i * D], zblk, wm[i * D:]], axis=0))
    return jnp.concatenate(cols, axis=1)


def kernel(x, edge_index, edge_weight, weight, root, bias, wa, ba, motif_w,
           motif_b):
    nchunk_tot = EPAD // CHUNK
    src = jnp.pad(edge_index[:, 0, :],
                  ((0, 0), (0, EPAD - E))).reshape(NG, nchunk_tot, CHUNK)
    dst = jnp.pad(edge_index[:, 1, :],
                  ((0, 0), (0, EPAD - E))).reshape(NG, nchunk_tot, CHUNK)
    ew = jnp.pad(edge_weight,
                 ((0, 0), (0, EPAD - E))).reshape(NG, nchunk_tot, CHUNK)

    p0 = _make_conv(1)(x, src[:1], dst[:1], ew[:1])[:, 0, :N, :]

    h = pl.pallas_call(
        _h_body,
        grid=(N // R_H,),
        in_specs=[
            pl.BlockSpec((NCORES, R_H, D), lambda i: (0, i, 0)),
            pl.BlockSpec((R_H, D), lambda i: (i, 0)),
            pl.BlockSpec((D, D), lambda i: (0, 0)),
            pl.BlockSpec((D, D), lambda i: (0, 0)),
            pl.BlockSpec((1, D), lambda i: (0, 0)),
        ],
        out_specs=pl.BlockSpec((R_H, D), lambda i: (i, 0)),
        out_shape=jax.ShapeDtypeStruct((N, D), jnp.float32),
    )(p0, x, weight, root, bias[None, :])

    p = _make_conv(NMOTIF)(h, src[1:], dst[1:], ew[1:])[:, :, :N, :]

    wbig = _build_wbig(motif_w)
    bcat = motif_b.reshape(1, NMOTIF * CD)
    batile = jnp.tile(ba, NMOTIF)[None, :]

    out = pl.pallas_call(
        _att_body,
        grid=(N // R_A,),
        in_specs=[
            pl.BlockSpec((R_A, D), lambda i: (i, 0)),
            pl.BlockSpec((NCORES, NMOTIF, R_A, D), lambda i: (0, 0, i, 0)),
            pl.BlockSpec(((NMOTIF + 1) * D, NMOTIF * CD), lambda i: (0, 0)),
            pl.BlockSpec((1, NMOTIF * CD), lambda i: (0, 0)),
            pl.BlockSpec((D, CD), lambda i: (0, 0)),
            pl.BlockSpec((1, NMOTIF * CD), lambda i: (0, 0)),
        ],
        out_specs=pl.BlockSpec((R_A, NMOTIF * CD), lambda i: (i, 0)),
        out_shape=jax.ShapeDtypeStruct((N, NMOTIF * CD), jnp.float32),
    )(h, p, wbig, bcat, wa, batile)
    return out


# speedup vs baseline: 2.6274x; 2.3286x over previous
import functools

import jax
import jax.numpy as jnp
from jax import lax
from jax.experimental import pallas as pl
from jax.experimental.pallas import tpu as pltpu
from jax.experimental.pallas import tpu_sc as plsc

N = 10000
D = 128
CD = 64
E = 320000
NMOTIF = 13
NG = NMOTIF + 1

NCORES = 2
NSUB = 16
NTILES = NCORES * NSUB
CHUNK = 128
RUN_CHUNKS = 8
PAIRS = RUN_CHUNKS // 2
RUNS = 10
NCHUNKS = RUNS * RUN_CHUNKS
EPAD = NCHUNKS * NTILES * CHUNK
NP = 10240
RPT = NP // NSUB


def _conv_body(ng, xx, src, dst, ew, out, src_b, dst_b, ew_b, rows0, rows1,
               zero_v, acc, sg0, sg1, ss0, ss1):
    cid = lax.axis_index("c")
    sid = lax.axis_index("s")
    cbase = (cid * NSUB + sid) * NCHUNKS

    z16 = jnp.zeros((16,), jnp.float32)
    for r in range(16):
        for c in range(D // 16):
            zero_v[r, pl.ds(c * 16, 16)] = z16

    def scale(rv, wrow):
        def group_body(gi, carry3):
            wg = ew_b[wrow, pl.ds(gi * 16, 16)]
            for t in range(16):
                e = gi * 16 + t
                w = lax.gather(
                    wg, jnp.full((16, 1), t, jnp.int32),
                    lax.GatherDimensionNumbers(
                        offset_dims=(), collapsed_slice_dims=(0,),
                        start_index_map=(0,)),
                    slice_sizes=(1,),
                    mode=lax.GatherScatterMode.PROMISE_IN_BOUNDS)
                for c in range(D // 16):
                    rv[e, pl.ds(c * 16, 16)] = rv[e, pl.ds(c * 16, 16)] * w
            return carry3

        lax.fori_loop(0, CHUNK // 16, group_body, 0)

    def graph_body(g, carry):
        def zero_body(z, carry0):
            pltpu.sync_copy(zero_v, acc.at[pl.ds(sid * RPT + z * 16, 16)])
            return carry0

        with jax.named_scope("acc_zero"):
            lax.fori_loop(0, RPT // 16, zero_body, 0)
            plsc.subcore_barrier()

        def run_body(r, carry1):
            rb = cbase + r * RUN_CHUNKS
            pltpu.sync_copy(src.at[g, pl.ds(rb, RUN_CHUNKS)], src_b)
            pltpu.sync_copy(dst.at[g, pl.ds(rb, RUN_CHUNKS)], dst_b)
            pltpu.sync_copy(ew.at[g, pl.ds(rb, RUN_CHUNKS)], ew_b)
            pltpu.async_copy(xx.at[src_b.at[0]], rows0, sg0)

            def pair_body(p, carry2):
                j0 = 2 * p
                j1 = 2 * p + 1

                @pl.when(p > 0)
                def _wait_s1():
                    pltpu.make_async_copy(rows1, acc.at[dst_b.at[j1]],
                                          ss1).wait()

                pltpu.async_copy(xx.at[src_b.at[j1]], rows1, sg1)
                pltpu.make_async_copy(xx.at[src_b.at[j0]], rows0, sg0).wait()
                scale(rows0, j0)
                pltpu.async_copy(rows0, acc.at[dst_b.at[j0]], ss0, add=True)
                pltpu.make_async_copy(xx.at[src_b.at[j1]], rows1, sg1).wait()
                scale(rows1, j1)
                pltpu.make_async_copy(rows0, acc.at[dst_b.at[j0]], ss0).wait()

                @pl.when(p < PAIRS - 1)
                def _next_g0():
                    pltpu.async_copy(xx.at[src_b.at[j0 + 2]], rows0, sg0)

                pltpu.async_copy(rows1, acc.at[dst_b.at[j1]], ss1, add=True)
                return carry2

            lax.fori_loop(0, PAIRS, pair_body, 0)
            pltpu.make_async_copy(rows1, acc.at[dst_b.at[RUN_CHUNKS - 1]],
                                  ss1).wait()
            return carry1

        with jax.named_scope("edges"):
            lax.fori_loop(0, RUNS, run_body, 0)
            plsc.subcore_barrier()
        with jax.named_scope("copyout"):
            pltpu.sync_copy(acc.at[pl.ds(sid * RPT, RPT)],
                            out.at[cid, g, pl.ds(sid * RPT, RPT)])
        return carry

    lax.fori_loop(0, ng, graph_body, 0)


@functools.lru_cache(maxsize=None)
def _make_conv(ng):
    mesh = plsc.VectorSubcoreMesh(core_axis_name="c", subcore_axis_name="s")
    return pl.kernel(
        functools.partial(_conv_body, ng),
        out_type=jax.ShapeDtypeStruct((NCORES, ng, NP, D), jnp.float32),
        mesh=mesh,
        scratch_types=[
            pltpu.VMEM((RUN_CHUNKS, CHUNK), jnp.int32),
            pltpu.VMEM((RUN_CHUNKS, CHUNK), jnp.int32),
            pltpu.VMEM((RUN_CHUNKS, CHUNK), jnp.float32),
            pltpu.VMEM((CHUNK, D), jnp.float32),
            pltpu.VMEM((CHUNK, D), jnp.float32),
            pltpu.VMEM((16, D), jnp.float32),
            pltpu.VMEM_SHARED((NP, D), jnp.float32),
            pltpu.SemaphoreType.DMA,
            pltpu.SemaphoreType.DMA,
            pltpu.SemaphoreType.DMA,
            pltpu.SemaphoreType.DMA,
        ],
    )


R_H = 1000


def _h_body(p_ref, x_ref, w_ref, r_ref, b_ref, o_ref):
    agg = p_ref[0] + p_ref[1]
    o_ref[...] = (
        jnp.dot(agg, w_ref[...], preferred_element_type=jnp.float32)
        + jnp.dot(x_ref[...], r_ref[...], preferred_element_type=jnp.float32)
        + b_ref[...]
    )


R_A = 200


def _att_body(h_ref, p_ref, wb_ref, bc_ref, wa_ref, ba_ref, o_ref):
    f32 = jnp.float32
    wa = wa_ref[...]
    c = jnp.dot(h_ref[...], wb_ref[0:D], preferred_element_type=f32)
    mws = []
    for j in range(NMOTIF):
        mj = p_ref[0, j] + p_ref[1, j]
        c = c + jnp.dot(mj, wb_ref[D * (j + 1):D * (j + 2)],
                        preferred_element_type=f32)
        mws.append(jnp.dot(mj, wa, preferred_element_type=f32))
    c = c + bc_ref[...]
    mw = jnp.concatenate(mws, axis=1) + ba_ref[...]
    att = jnp.tanh(jnp.sum((mw * c).reshape(R_A, NMOTIF, CD), axis=2))
    diff = (mw - c).reshape(R_A, NMOTIF, CD)
    o_ref[...] = (att[:, :, None] * diff).reshape(R_A, NMOTIF * CD)


def _build_wbig(motif_w):
    cols = []
    zblk = jnp.zeros((D, CD), jnp.float32)
    for i in range(1, NMOTIF + 1):
        wm = motif_w[i - 1]
        cols.append(jnp.concatenate([wm[: i * D], zblk, wm[i * D:]], axis=0))
    return jnp.concatenate(cols, axis=1)


def kernel(x, edge_index, edge_weight, weight, root, bias, wa, ba, motif_w,
           motif_b):
    nchunk_tot = EPAD // CHUNK
    pad_idx = jnp.broadcast_to(
        jnp.arange(EPAD - E, dtype=jnp.int32) % N, (NG, EPAD - E))
    src = jnp.concatenate([edge_index[:, 0, :], pad_idx],
                          axis=1).reshape(NG, nchunk_tot, CHUNK)
    dst = jnp.concatenate([edge_index[:, 1, :], pad_idx],
                          axis=1).reshape(NG, nchunk_tot, CHUNK)
    ew = jnp.pad(edge_weight,
                 ((0, 0), (0, EPAD - E))).reshape(NG, nchunk_tot, CHUNK)

    p0 = _make_conv(1)(x, src[:1], dst[:1], ew[:1])[:, 0, :N, :]

    h = pl.pallas_call(
        _h_body,
        grid=(N // R_H,),
        in_specs=[
            pl.BlockSpec((NCORES, R_H, D), lambda i: (0, i, 0)),
            pl.BlockSpec((R_H, D), lambda i: (i, 0)),
            pl.BlockSpec((D, D), lambda i: (0, 0)),
            pl.BlockSpec((D, D), lambda i: (0, 0)),
            pl.BlockSpec((1, D), lambda i: (0, 0)),
        ],
        out_specs=pl.BlockSpec((R_H, D), lambda i: (i, 0)),
        out_shape=jax.ShapeDtypeStruct((N, D), jnp.float32),
    )(p0, x, weight, root, bias[None, :])

    p = _make_conv(NMOTIF)(h, src[1:], dst[1:], ew[1:])[:, :, :N, :]

    wbig = _build_wbig(motif_w)
    bcat = motif_b.reshape(1, NMOTIF * CD)
    batile = jnp.tile(ba, NMOTIF)[None, :]

    out = pl.pallas_call(
        _att_body,
        grid=(N // R_A,),
        in_specs=[
            pl.BlockSpec((R_A, D), lambda i: (i, 0)),
            pl.BlockSpec((NCORES, NMOTIF, R_A, D), lambda i: (0, 0, i, 0)),
            pl.BlockSpec(((NMOTIF + 1) * D, NMOTIF * CD), lambda i: (0, 0)),
            pl.BlockSpec((1, NMOTIF * CD), lambda i: (0, 0)),
            pl.BlockSpec((D, CD), lambda i: (0, 0)),
            pl.BlockSpec((1, NMOTIF * CD), lambda i: (0, 0)),
        ],
        out_specs=pl.BlockSpec((R_A, NMOTIF * CD), lambda i: (i, 0)),
        out_shape=jax.ShapeDtypeStruct((N, NMOTIF * CD), jnp.float32),
    )(h, p, wbig, bcat, wa, batile)
    return out


# speedup vs baseline: 2.9670x; 1.1292x over previous
import functools

import jax
import jax.numpy as jnp
from jax import lax
from jax.experimental import pallas as pl
from jax.experimental.pallas import tpu as pltpu
from jax.experimental.pallas import tpu_sc as plsc

N = 10000
D = 128
CD = 64
E = 320000
NMOTIF = 13
NG = NMOTIF + 1

NCORES = 2
NSUB = 16
NTILES = NCORES * NSUB
CHUNK = 128
RUN_CHUNKS = 8
PAIRS = RUN_CHUNKS // 2
RUNS = 10
NCHUNKS = RUNS * RUN_CHUNKS
EPAD = NCHUNKS * NTILES * CHUNK
NP = 10240
RPT = NP // NSUB


def _conv_body(ng, xx, edata, ewd, out, ed0, ed1, ew0, ew1, rows0, rows1,
               zero_v, acc, sg0, sg1, ss0, ss1, si0, si1):
    cid = lax.axis_index("c")
    sid = lax.axis_index("s")
    cbase = (cid * NSUB + sid) * NCHUNKS

    z16 = jnp.zeros((16,), jnp.float32)
    for r in range(16):
        for c in range(D // 16):
            zero_v[r, pl.ds(c * 16, 16)] = z16

    def zero_body(z, carry0):
        pltpu.sync_copy(zero_v, acc.at[pl.ds(sid * RPT + z * 16, 16)])
        return carry0

    lax.fori_loop(0, RPT // 16, zero_body, 0)
    plsc.subcore_barrier()

    def scale(rv, ewb, j):
        def group_body(gi, carry3):
            wg = ewb[j, pl.ds(gi * 16, 16)]
            for t in range(16):
                e = gi * 16 + t
                w = lax.gather(
                    wg, jnp.full((16, 1), t, jnp.int32),
                    lax.GatherDimensionNumbers(
                        offset_dims=(), collapsed_slice_dims=(0,),
                        start_index_map=(0,)),
                    slice_sizes=(1,),
                    mode=lax.GatherScatterMode.PROMISE_IN_BOUNDS)
                for c in range(D // 16):
                    rv[e, pl.ds(c * 16, 16)] = rv[e, pl.ds(c * 16, 16)] * w
            return carry3

        lax.fori_loop(0, CHUNK // 16, group_body, 0)

    def process_run(ed, ewb):
        pltpu.async_copy(xx.at[ed.at[0, 0]], rows0, sg0)

        def pair_body(p, carry2):
            j0 = 2 * p
            j1 = 2 * p + 1

            @pl.when(p > 0)
            def _wait_s1():
                pltpu.make_async_copy(rows1, acc.at[ed.at[j1, 1]], ss1).wait()

            pltpu.async_copy(xx.at[ed.at[j1, 0]], rows1, sg1)
            pltpu.make_async_copy(xx.at[ed.at[j0, 0]], rows0, sg0).wait()
            scale(rows0, ewb, j0)
            pltpu.async_copy(rows0, acc.at[ed.at[j0, 1]], ss0, add=True)
            pltpu.make_async_copy(xx.at[ed.at[j1, 0]], rows1, sg1).wait()
            scale(rows1, ewb, j1)
            pltpu.make_async_copy(rows0, acc.at[ed.at[j0, 1]], ss0).wait()

            @pl.when(p < PAIRS - 1)
            def _next_g0():
                pltpu.async_copy(xx.at[ed.at[j0 + 2, 0]], rows0, sg0)

            pltpu.async_copy(rows1, acc.at[ed.at[j1, 1]], ss1, add=True)
            return carry2

        lax.fori_loop(0, PAIRS, pair_body, 0)
        pltpu.make_async_copy(rows1, acc.at[ed.at[RUN_CHUNKS - 1, 1]],
                              ss1).wait()

    def load_idx(g, rb, ed, ewb, sem):
        pltpu.async_copy(edata.at[g, pl.ds(rb, RUN_CHUNKS)], ed, sem)
        pltpu.async_copy(ewd.at[g, pl.ds(rb, RUN_CHUNKS)], ewb, sem)

    def wait_idx(g, rb, ed, ewb, sem):
        pltpu.make_async_copy(edata.at[g, pl.ds(rb, RUN_CHUNKS)], ed,
                              sem).wait()
        pltpu.make_async_copy(ewd.at[g, pl.ds(rb, RUN_CHUNKS)], ewb,
                              sem).wait()

    def graph_body(g, carry):
        with jax.named_scope("edges"):
            load_idx(g, cbase, ed0, ew0, si0)
            wait_idx(g, cbase, ed0, ew0, si0)

            def rpair_body(q, carry1):
                rb = cbase + 2 * q * RUN_CHUNKS
                load_idx(g, rb + RUN_CHUNKS, ed1, ew1, si1)
                process_run(ed0, ew0)
                wait_idx(g, rb + RUN_CHUNKS, ed1, ew1, si1)

                @pl.when(q < RUNS // 2 - 1)
                def _pref0():
                    load_idx(g, rb + 2 * RUN_CHUNKS, ed0, ew0, si0)

                process_run(ed1, ew1)

                @pl.when(q < RUNS // 2 - 1)
                def _wait0():
                    wait_idx(g, rb + 2 * RUN_CHUNKS, ed0, ew0, si0)

                return carry1

            lax.fori_loop(0, RUNS // 2, rpair_body, 0)
            plsc.subcore_barrier()
        with jax.named_scope("copyout"):
            pltpu.sync_copy(acc.at[pl.ds(sid * RPT, RPT)],
                            out.at[cid, g, pl.ds(sid * RPT, RPT)])
        plsc.subcore_barrier()
        return carry

    lax.fori_loop(0, ng, graph_body, 0)


@functools.lru_cache(maxsize=None)
def _make_conv(ng):
    mesh = plsc.VectorSubcoreMesh(core_axis_name="c", subcore_axis_name="s")
    return pl.kernel(
        functools.partial(_conv_body, ng),
        out_type=jax.ShapeDtypeStruct((NCORES, ng, NP, D), jnp.float32),
        mesh=mesh,
        scratch_types=[
            pltpu.VMEM((RUN_CHUNKS, 2, CHUNK), jnp.int32),
            pltpu.VMEM((RUN_CHUNKS, 2, CHUNK), jnp.int32),
            pltpu.VMEM((RUN_CHUNKS, CHUNK), jnp.float32),
            pltpu.VMEM((RUN_CHUNKS, CHUNK), jnp.float32),
            pltpu.VMEM((CHUNK, D), jnp.float32),
            pltpu.VMEM((CHUNK, D), jnp.float32),
            pltpu.VMEM((16, D), jnp.float32),
            pltpu.VMEM_SHARED((NP, D), jnp.float32),
            pltpu.SemaphoreType.DMA,
            pltpu.SemaphoreType.DMA,
            pltpu.SemaphoreType.DMA,
            pltpu.SemaphoreType.DMA,
            pltpu.SemaphoreType.DMA,
            pltpu.SemaphoreType.DMA,
        ],
    )


R_H = 1000


def _h_body(p_ref, x_ref, w_ref, r_ref, b_ref, o_ref):
    agg = p_ref[0] + p_ref[1]
    o_ref[...] = (
        jnp.dot(agg, w_ref[...], preferred_element_type=jnp.float32)
        + jnp.dot(x_ref[...], r_ref[...], preferred_element_type=jnp.float32)
        + b_ref[...]
    )


R_A = 200


def _att_body(h_ref, p_ref, wb_ref, bc_ref, wa_ref, ba_ref, o_ref):
    f32 = jnp.float32
    wa = wa_ref[...]
    c = jnp.dot(h_ref[...], wb_ref[0:D], preferred_element_type=f32)
    mws = []
    prev = None
    for j in range(NMOTIF):
        cum = p_ref[0, j] + p_ref[1, j]
        mj = cum if prev is None else cum - prev
        prev = cum
        c = c + jnp.dot(mj, wb_ref[D * (j + 1):D * (j + 2)],
                        preferred_element_type=f32)
        mws.append(jnp.dot(mj, wa, preferred_element_type=f32))
    c = c + bc_ref[...]
    mw = jnp.concatenate(mws, axis=1) + ba_ref[...]
    att = jnp.tanh(jnp.sum((mw * c).reshape(R_A, NMOTIF, CD), axis=2))
    diff = (mw - c).reshape(R_A, NMOTIF, CD)
    o_ref[...] = (att[:, :, None] * diff).reshape(R_A, NMOTIF * CD)


def _build_wbig(motif_w):
    cols = []
    zblk = jnp.zeros((D, CD), jnp.float32)
    for i in range(1, NMOTIF + 1):
        wm = motif_w[i - 1]
        cols.append(jnp.concatenate([wm[: i * D], zblk, wm[i * D:]], axis=0))
    return jnp.concatenate(cols, axis=1)


def kernel(x, edge_index, edge_weight, weight, root, bias, wa, ba, motif_w,
           motif_b):
    nchunk_tot = EPAD // CHUNK
    pad_idx = jnp.broadcast_to(
        jnp.arange(EPAD - E, dtype=jnp.int32) % N, (NG, EPAD - E))
    src = jnp.concatenate([edge_index[:, 0, :], pad_idx], axis=1)
    dst = jnp.concatenate([edge_index[:, 1, :], pad_idx], axis=1)
    edata = jnp.stack(
        [src.reshape(NG, nchunk_tot, CHUNK),
         dst.reshape(NG, nchunk_tot, CHUNK)], axis=2)
    ewd = jnp.pad(edge_weight,
                  ((0, 0), (0, EPAD - E))).reshape(NG, nchunk_tot, CHUNK)

    p0 = _make_conv(1)(x, edata[:1], ewd[:1]).reshape(NCORES, NP, D)

    h = pl.pallas_call(
        _h_body,
        grid=(N // R_H,),
        in_specs=[
            pl.BlockSpec((NCORES, R_H, D), lambda i: (0, i, 0)),
            pl.BlockSpec((R_H, D), lambda i: (i, 0)),
            pl.BlockSpec((D, D), lambda i: (0, 0)),
            pl.BlockSpec((D, D), lambda i: (0, 0)),
            pl.BlockSpec((1, D), lambda i: (0, 0)),
        ],
        out_specs=pl.BlockSpec((R_H, D), lambda i: (i, 0)),
        out_shape=jax.ShapeDtypeStruct((N, D), jnp.float32),
    )(p0, x, weight, root, bias[None, :])

    p = _make_conv(NMOTIF)(h, edata[1:], ewd[1:])

    wbig = _build_wbig(motif_w)
    bcat = motif_b.reshape(1, NMOTIF * CD)
    batile = jnp.tile(ba, NMOTIF)[None, :]

    out = pl.pallas_call(
        _att_body,
        grid=(N // R_A,),
        in_specs=[
            pl.BlockSpec((R_A, D), lambda i: (i, 0)),
            pl.BlockSpec((NCORES, NMOTIF, R_A, D), lambda i: (0, 0, i, 0)),
            pl.BlockSpec(((NMOTIF + 1) * D, NMOTIF * CD), lambda i: (0, 0)),
            pl.BlockSpec((1, NMOTIF * CD), lambda i: (0, 0)),
            pl.BlockSpec((D, CD), lambda i: (0, 0)),
            pl.BlockSpec((1, NMOTIF * CD), lambda i: (0, 0)),
        ],
        out_specs=pl.BlockSpec((R_A, NMOTIF * CD), lambda i: (i, 0)),
        out_shape=jax.ShapeDtypeStruct((N, NMOTIF * CD), jnp.float32),
    )(h, p, wbig, bcat, wa, batile)
    return out


# speedup vs baseline: 3.0463x; 1.0267x over previous
import functools

import jax
import jax.numpy as jnp
from jax import lax
from jax.experimental import pallas as pl
from jax.experimental.pallas import tpu as pltpu
from jax.experimental.pallas import tpu_sc as plsc

N = 10000
D = 128
CD = 64
E = 320000
NMOTIF = 13
NG = NMOTIF + 1

NCORES = 2
NSUB = 16
NTILES = NCORES * NSUB
CHUNK = 128
RUN_CHUNKS = 8
PAIRS = RUN_CHUNKS // 2
RUNS = 10
NCHUNKS = RUNS * RUN_CHUNKS
EPAD = NCHUNKS * NTILES * CHUNK
NP = 10240
RPT = NP // NSUB


def _conv_body(ng, xx, srcd, dstd, ewd, out, sb0, sb1, db0, db1, ew0, ew1,
               rows0, rows1, zero_v, acc, sg0, sg1, ss0, ss1, si0, si1):
    cid = lax.axis_index("c")
    sid = lax.axis_index("s")
    cbase = (cid * NSUB + sid) * NCHUNKS

    z16 = jnp.zeros((16,), jnp.float32)
    for r in range(16):
        for c in range(D // 16):
            zero_v[r, pl.ds(c * 16, 16)] = z16

    def zero_body(z, carry0):
        pltpu.sync_copy(zero_v, acc.at[pl.ds(sid * RPT + z * 16, 16)])
        return carry0

    lax.fori_loop(0, RPT // 16, zero_body, 0)
    plsc.subcore_barrier()

    def scale(rv, ewb, j):
        def group_body(gi, carry3):
            wg = ewb[j, pl.ds(gi * 16, 16)]
            for t in range(16):
                e = gi * 16 + t
                w = lax.gather(
                    wg, jnp.full((16, 1), t, jnp.int32),
                    lax.GatherDimensionNumbers(
                        offset_dims=(), collapsed_slice_dims=(0,),
                        start_index_map=(0,)),
                    slice_sizes=(1,),
                    mode=lax.GatherScatterMode.PROMISE_IN_BOUNDS)
                for c in range(D // 16):
                    rv[e, pl.ds(c * 16, 16)] = rv[e, pl.ds(c * 16, 16)] * w
            return carry3

        lax.fori_loop(0, CHUNK // 16, group_body, 0)

    def process_run(sb, db, ewb):
        pltpu.async_copy(xx.at[sb.at[0]], rows0, sg0)

        def pair_body(p, carry2):
            j0 = 2 * p
            j1 = 2 * p + 1

            @pl.when(p > 0)
            def _wait_s1():
                pltpu.make_async_copy(rows1, acc.at[db.at[j1]], ss1).wait()

            pltpu.async_copy(xx.at[sb.at[j1]], rows1, sg1)
            pltpu.make_async_copy(xx.at[sb.at[j0]], rows0, sg0).wait()
            scale(rows0, ewb, j0)
            pltpu.async_copy(rows0, acc.at[db.at[j0]], ss0, add=True)
            pltpu.make_async_copy(xx.at[sb.at[j1]], rows1, sg1).wait()
            scale(rows1, ewb, j1)
            pltpu.make_async_copy(rows0, acc.at[db.at[j0]], ss0).wait()

            @pl.when(p < PAIRS - 1)
            def _next_g0():
                pltpu.async_copy(xx.at[sb.at[j0 + 2]], rows0, sg0)

            pltpu.async_copy(rows1, acc.at[db.at[j1]], ss1, add=True)
            return carry2

        lax.fori_loop(0, PAIRS, pair_body, 0)
        pltpu.make_async_copy(rows1, acc.at[db.at[RUN_CHUNKS - 1]],
                              ss1).wait()

    def load_idx(g, rb, sb, db, ewb, sem, wait):
        for hbm, buf in ((srcd, sb), (dstd, db), (ewd, ewb)):
            cp = pltpu.make_async_copy(hbm.at[g, pl.ds(rb, RUN_CHUNKS)],
                                       buf, sem)
            if wait:
                cp.wait()
            else:
                cp.start()

    def graph_body(g, carry):
        with jax.named_scope("edges"):
            load_idx(g, cbase, sb0, db0, ew0, si0, False)
            load_idx(g, cbase, sb0, db0, ew0, si0, True)

            def rpair_body(q, carry1):
                rb = cbase + 2 * q * RUN_CHUNKS
                load_idx(g, rb + RUN_CHUNKS, sb1, db1, ew1, si1, False)
                process_run(sb0, db0, ew0)
                load_idx(g, rb + RUN_CHUNKS, sb1, db1, ew1, si1, True)

                @pl.when(q < RUNS // 2 - 1)
                def _pref0():
                    load_idx(g, rb + 2 * RUN_CHUNKS, sb0, db0, ew0, si0,
                             False)

                process_run(sb1, db1, ew1)

                @pl.when(q < RUNS // 2 - 1)
                def _wait0():
                    load_idx(g, rb + 2 * RUN_CHUNKS, sb0, db0, ew0, si0,
                             True)

                return carry1

            lax.fori_loop(0, RUNS // 2, rpair_body, 0)
            plsc.subcore_barrier()
        with jax.named_scope("copyout"):
            pltpu.sync_copy(acc.at[pl.ds(sid * RPT, RPT)],
                            out.at[cid, g, pl.ds(sid * RPT, RPT)])
        plsc.subcore_barrier()
        return carry

    lax.fori_loop(0, ng, graph_body, 0)


@functools.lru_cache(maxsize=None)
def _make_conv(ng):
    mesh = plsc.VectorSubcoreMesh(core_axis_name="c", subcore_axis_name="s")
    return pl.kernel(
        functools.partial(_conv_body, ng),
        out_type=jax.ShapeDtypeStruct((NCORES, ng, NP, D), jnp.float32),
        mesh=mesh,
        scratch_types=[
            pltpu.VMEM((RUN_CHUNKS, CHUNK), jnp.int32),
            pltpu.VMEM((RUN_CHUNKS, CHUNK), jnp.int32),
            pltpu.VMEM((RUN_CHUNKS, CHUNK), jnp.int32),
            pltpu.VMEM((RUN_CHUNKS, CHUNK), jnp.int32),
            pltpu.VMEM((RUN_CHUNKS, CHUNK), jnp.float32),
            pltpu.VMEM((RUN_CHUNKS, CHUNK), jnp.float32),
            pltpu.VMEM((CHUNK, D), jnp.float32),
            pltpu.VMEM((CHUNK, D), jnp.float32),
            pltpu.VMEM((16, D), jnp.float32),
            pltpu.VMEM_SHARED((NP, D), jnp.float32),
            pltpu.SemaphoreType.DMA,
            pltpu.SemaphoreType.DMA,
            pltpu.SemaphoreType.DMA,
            pltpu.SemaphoreType.DMA,
            pltpu.SemaphoreType.DMA,
            pltpu.SemaphoreType.DMA,
        ],
    )


R_H = 1000


def _h_body(p_ref, x_ref, w_ref, r_ref, b_ref, o_ref):
    agg = p_ref[0] + p_ref[1]
    o_ref[...] = (
        jnp.dot(agg, w_ref[...], preferred_element_type=jnp.float32)
        + jnp.dot(x_ref[...], r_ref[...], preferred_element_type=jnp.float32)
        + b_ref[...]
    )


R_A = 200


def _att_body(h_ref, p_ref, wb_ref, bc_ref, wa_ref, ba_ref, o_ref):
    f32 = jnp.float32
    wa = wa_ref[...]
    c = jnp.dot(h_ref[...], wb_ref[0:D], preferred_element_type=f32)
    mws = []
    prev = None
    for j in range(NMOTIF):
        cum = p_ref[0, j] + p_ref[1, j]
        mj = cum if prev is None else cum - prev
        prev = cum
        c = c + jnp.dot(mj, wb_ref[D * (j + 1):D * (j + 2)],
                        preferred_element_type=f32)
        mws.append(jnp.dot(mj, wa, preferred_element_type=f32))
    c = c + bc_ref[...]
    mw = jnp.concatenate(mws, axis=1) + ba_ref[...]
    att = jnp.tanh(jnp.sum((mw * c).reshape(R_A, NMOTIF, CD), axis=2))
    diff = (mw - c).reshape(R_A, NMOTIF, CD)
    o_ref[...] = (att[:, :, None] * diff).reshape(R_A, NMOTIF * CD)


def _build_wbig(motif_w):
    cols = []
    zblk = jnp.zeros((D, CD), jnp.float32)
    for i in range(1, NMOTIF + 1):
        wm = motif_w[i - 1]
        cols.append(jnp.concatenate([wm[: i * D], zblk, wm[i * D:]], axis=0))
    return jnp.concatenate(cols, axis=1)


def kernel(x, edge_index, edge_weight, weight, root, bias, wa, ba, motif_w,
           motif_b):
    nchunk_tot = EPAD // CHUNK
    pad_idx = jnp.broadcast_to(
        jnp.arange(EPAD - E, dtype=jnp.int32) % N, (NG, EPAD - E))
    src = jnp.concatenate([edge_index[:, 0, :], pad_idx],
                          axis=1).reshape(NG, nchunk_tot, CHUNK)
    dst = jnp.concatenate([edge_index[:, 1, :], pad_idx],
                          axis=1).reshape(NG, nchunk_tot, CHUNK)
    ewd = jnp.pad(edge_weight,
                  ((0, 0), (0, EPAD - E))).reshape(NG, nchunk_tot, CHUNK)

    p0 = _make_conv(1)(x, src[:1], dst[:1], ewd[:1]).reshape(NCORES, NP, D)

    h = pl.pallas_call(
        _h_body,
        grid=(N // R_H,),
        in_specs=[
            pl.BlockSpec((NCORES, R_H, D), lambda i: (0, i, 0)),
            pl.BlockSpec((R_H, D), lambda i: (i, 0)),
            pl.BlockSpec((D, D), lambda i: (0, 0)),
            pl.BlockSpec((D, D), lambda i: (0, 0)),
            pl.BlockSpec((1, D), lambda i: (0, 0)),
        ],
        out_specs=pl.BlockSpec((R_H, D), lambda i: (i, 0)),
        out_shape=jax.ShapeDtypeStruct((N, D), jnp.float32),
    )(p0, x, weight, root, bias[None, :])

    p = _make_conv(NMOTIF)(h, src[1:], dst[1:], ewd[1:])

    wbig = _build_wbig(motif_w)
    bcat = motif_b.reshape(1, NMOTIF * CD)
    batile = jnp.tile(ba, NMOTIF)[None, :]

    out = pl.pallas_call(
        _att_body,
        grid=(N // R_A,),
        in_specs=[
            pl.BlockSpec((R_A, D), lambda i: (i, 0)),
            pl.BlockSpec((NCORES, NMOTIF, R_A, D), lambda i: (0, 0, i, 0)),
            pl.BlockSpec(((NMOTIF + 1) * D, NMOTIF * CD), lambda i: (0, 0)),
            pl.BlockSpec((1, NMOTIF * CD), lambda i: (0, 0)),
            pl.BlockSpec((D, CD), lambda i: (0, 0)),
            pl.BlockSpec((1, NMOTIF * CD), lambda i: (0, 0)),
        ],
        out_specs=pl.BlockSpec((R_A, NMOTIF * CD), lambda i: (i, 0)),
        out_shape=jax.ShapeDtypeStruct((N, NMOTIF * CD), jnp.float32),
    )(h, p, wbig, bcat, wa, batile)
    return out
